# baseline (speedup 1.0000x reference)
import functools
import math

import jax
import jax.numpy as jnp
from jax import lax
from jax.experimental import pallas as pl
from jax.experimental.pallas import tpu as pltpu
from jax.experimental.pallas import tpu_sc as plsc

N = 10000
D = 256
E = 160000
NET = 2
NPAD = 10240
HALF = 128
ET = E + N
K = 64
CPT = 168
EPT = CPT * K
EP = 16 * EPT
EPT32 = EP // 32
ROWS_PT = NPAD // 16
RED = 2 * NPAD // 16

ALPHA = 0.1
BETA1 = math.log(2.0)
BETA2 = math.log(1.5)

BN = 512
GRID = NPAD // BN


def _deg_body(src_ref, dst_ref, out_ref, acc, sbuf, dbuf, red, res, stage):
    c = lax.axis_index("c")
    s = lax.axis_index("s")
    w = c * 16 + s
    zeros16 = jnp.zeros((16,), jnp.int32)
    ones16 = jnp.ones((16,), jnp.int32)

    @pl.loop(0, 2 * NPAD // 16)
    def _zero(i):
        acc[pl.ds(i * 16, 16)] = zeros16

    pltpu.sync_copy(src_ref.at[w], sbuf)
    pltpu.sync_copy(dst_ref.at[w], dbuf)

    @pl.loop(0, EPT32 // 16)
    def _count(i):
        sv = sbuf[pl.ds(i * 16, 16)]
        plsc.addupdate_scatter(acc, [sv], ones16)
        dv = dbuf[pl.ds(i * 16, 16)] + NPAD
        plsc.addupdate_scatter(acc, [dv], ones16)

    pltpu.sync_copy(acc, stage.at[s])
    plsc.subcore_barrier()
    for k in range(16):
        pltpu.sync_copy(stage.at[k, pl.ds(s * RED, RED)], red.at[k])

    @pl.loop(0, RED // 16)
    def _reduce(j):
        tot = red[0, pl.ds(j * 16, 16)]
        for k in range(1, 16):
            tot = tot + red[k, pl.ds(j * 16, 16)]
        res[pl.ds(j * 16, 16)] = tot

    pltpu.sync_copy(res, out_ref.at[c, pl.ds(s * RED, RED)])


def _prop_body(table_ref, packed_ref, zeros_ref, out_ref,
               pbuf, gixs, dixs, rowbufs, acc, sems):
    c = lax.axis_index("c")
    s = lax.axis_index("s")
    coff = c * (table_ref.shape[0] // 2)
    pltpu.sync_copy(zeros_ref, acc.at[pl.ds(s * ROWS_PT, ROWS_PT)])
    pltpu.sync_copy(packed_ref.at[s], pbuf)
    plsc.subcore_barrier()

    def unpack(i, gb, db):
        for j in range(K // 16):
            pv = pbuf[pl.ds(i * K + j * 16, 16)]
            gb[pl.ds(j * 16, 16)] = (pv & 0xFFFF) + coff
            db[pl.ds(j * 16, 16)] = lax.shift_right_logical(pv, 16)

    @pl.loop(0, CPT // 4)
    def _edges(k):
        i = 4 * k
        descs = []
        for b in range(4):
            unpack(i + b, gixs[b], dixs[b])
            descs.append(
                pltpu.async_copy(table_ref.at[gixs[b]], rowbufs[b], sems[b]))
        for b in range(4):
            descs[b].wait()
            pltpu.sync_copy(rowbufs[b], acc.at[dixs[b]], add=True)

    plsc.subcore_barrier()
    pltpu.sync_copy(acc.at[pl.ds(s * ROWS_PT, ROWS_PT)],
                    out_ref.at[pl.ds(c * NPAD + s * ROWS_PT, ROWS_PT)])


@functools.lru_cache(maxsize=1)
def _sc_kernels():
    mesh = plsc.VectorSubcoreMesh(core_axis_name="c", subcore_axis_name="s")
    params = pltpu.CompilerParams(needs_layout_passes=False)
    deg = pl.kernel(
        _deg_body,
        out_type=jax.ShapeDtypeStruct((2, 2 * NPAD), jnp.int32),
        mesh=mesh,
        compiler_params=params,
        scratch_types=[
            pltpu.VMEM((2 * NPAD,), jnp.int32),
            pltpu.VMEM((EPT32,), jnp.int32),
            pltpu.VMEM((EPT32,), jnp.int32),
            pltpu.VMEM((16, RED), jnp.int32),
            pltpu.VMEM((RED,), jnp.int32),
            pltpu.VMEM_SHARED((16, 2 * NPAD), jnp.int32),
        ],
    )
    prop = pl.kernel(
        _prop_body,
        out_type=jax.ShapeDtypeStruct((2 * NPAD, HALF), jnp.float32),
        mesh=mesh,
        compiler_params=params,
        scratch_types=[
            pltpu.VMEM((EPT,), jnp.int32),
            [pltpu.VMEM((K,), jnp.int32) for _ in range(4)],
            [pltpu.VMEM((K,), jnp.int32) for _ in range(4)],
            [pltpu.VMEM((K, HALF), jnp.float32) for _ in range(4)],
            pltpu.VMEM_SHARED((NPAD, HALF), jnp.float32),
            [pltpu.SemaphoreType.DMA for _ in range(4)],
        ],
    )
    return deg, prop


def _norm_body(d_ref, out_ref):
    t = (d_ref[0] + d_ref[1]).astype(jnp.float32)
    r = lax.rsqrt(jnp.maximum(t, 1.0))
    out_ref[0] = r[:80]
    out_ref[1] = r[80:]


def _norm_call(deg2):
    return pl.pallas_call(
        _norm_body,
        out_shape=jax.ShapeDtypeStruct((2, 80, 128), jnp.float32),
    )(deg2.reshape(2, 160, 128))


def _scale_body(h_ref, n_ref, out_ref):
    h = h_ref[...]
    nv = n_ref[...]
    out_ref[0] = h[:, :HALF] * nv
    out_ref[1] = h[:, HALF:] * nv


def _scale_call(h_p, norm):
    return pl.pallas_call(
        _scale_body,
        grid=(GRID,),
        in_specs=[pl.BlockSpec((BN, D), lambda i: (i, 0)),
                  pl.BlockSpec((BN, 1), lambda i: (i, 0))],
        out_specs=pl.BlockSpec((2, BN, HALF), lambda i: (0, i, 0)),
        out_shape=jax.ShapeDtypeStruct((2, NPAD, HALF), jnp.float32),
    )(h_p, norm)


def _gcn2_body(a_ref, h0_ref, n_ref, w_ref, out_ref, *, beta):
    a = jnp.concatenate([a_ref[0], a_ref[1]], axis=1)
    hp = a * n_ref[...]
    f = (1.0 - ALPHA) * hp + ALPHA * h0_ref[...]
    rst = (1.0 - beta) * f + beta * jnp.dot(
        f, w_ref[...], preferred_element_type=jnp.float32)
    out_ref[...] = jnp.where(rst > 0, rst, jnp.exp(jnp.minimum(rst, 0.0)) - 1.0)


def _gcn2_call(agg3, h0_p, norm_in, W, beta):
    return pl.pallas_call(
        functools.partial(_gcn2_body, beta=beta),
        grid=(GRID,),
        in_specs=[pl.BlockSpec((2, BN, HALF), lambda i: (0, i, 0)),
                  pl.BlockSpec((BN, D), lambda i: (i, 0)),
                  pl.BlockSpec((BN, 1), lambda i: (i, 0)),
                  pl.BlockSpec((D, D), lambda i: (0, 0))],
        out_specs=pl.BlockSpec((BN, D), lambda i: (i, 0)),
        out_shape=jax.ShapeDtypeStruct((NPAD, D), jnp.float32),
    )(agg3, h0_p, norm_in, W)


def _trans_body(h_ref, w_ref, b_ref, out_ref):
    h = h_ref[...]
    for t in range(NET):
        m = lax.dot_general(h, w_ref[t], (((1,), (1,)), ((), ())),
                            preferred_element_type=jnp.float32) + b_ref[t]
        out_ref[0, t] = m[:, :HALF]
        out_ref[1, t] = m[:, HALF:]


def _trans_call(h_p, Wl, bl):
    return pl.pallas_call(
        _trans_body,
        grid=(GRID,),
        in_specs=[pl.BlockSpec((BN, D), lambda i: (i, 0)),
                  pl.BlockSpec((NET, D, D), lambda i: (0, 0, 0)),
                  pl.BlockSpec((NET, 1, D), lambda i: (0, 0, 0))],
        out_specs=pl.BlockSpec((2, NET, BN, HALF), lambda i: (0, 0, i, 0)),
        out_shape=jax.ShapeDtypeStruct((2, NET, NPAD, HALF), jnp.float32),
    )(h_p, Wl, bl.reshape(NET, 1, D))


def _gru_body(a_ref, h_ref, wih_ref, whh_ref, bih_ref, bhh_ref, out_ref):
    a = jnp.concatenate([a_ref[0], a_ref[1]], axis=1)
    h = h_ref[...]
    gi = lax.dot_general(a, wih_ref[...], (((1,), (1,)), ((), ())),
                         preferred_element_type=jnp.float32) + bih_ref[...]
    gh = lax.dot_general(h, whh_ref[...], (((1,), (1,)), ((), ())),
                         preferred_element_type=jnp.float32) + bhh_ref[...]
    r = jax.nn.sigmoid(gi[:, :D] + gh[:, :D])
    z = jax.nn.sigmoid(gi[:, D:2 * D] + gh[:, D:2 * D])
    g = jnp.tanh(gi[:, 2 * D:] + r * gh[:, 2 * D:])
    out_ref[...] = (1.0 - z) * g + z * h


def _gru_call(a3, h_p, Wih, Whh, bih, bhh):
    return pl.pallas_call(
        _gru_body,
        grid=(GRID,),
        in_specs=[pl.BlockSpec((2, BN, HALF), lambda i: (0, i, 0)),
                  pl.BlockSpec((BN, D), lambda i: (i, 0)),
                  pl.BlockSpec((3 * D, D), lambda i: (0, 0)),
                  pl.BlockSpec((3 * D, D), lambda i: (0, 0)),
                  pl.BlockSpec((1, 3 * D), lambda i: (0, 0)),
                  pl.BlockSpec((1, 3 * D), lambda i: (0, 0))],
        out_specs=pl.BlockSpec((BN, D), lambda i: (i, 0)),
        out_shape=jax.ShapeDtypeStruct((NPAD, D), jnp.float32),
    )(a3, h_p, Wih, Whh, bih.reshape(1, 3 * D), bhh.reshape(1, 3 * D))


def kernel(feats, edge_index, etypes, W_gcn_0, W_lin_0, b_lin_0, W_ih_0,
           W_hh_0, b_ih_0, b_hh_0, W_gcn_1, W_lin_1, b_lin_1, W_ih_1,
           W_hh_1, b_ih_1, b_hh_1):
    i32 = jnp.int32
    loop = jnp.arange(N, dtype=i32)
    src = jnp.concatenate([edge_index[0], loop])
    dst = jnp.concatenate([edge_index[1], loop])
    et = jnp.concatenate([etypes, jnp.zeros((N,), dtype=etypes.dtype)])
    npad_edges = EP - ET
    trash = NPAD - 1
    src_p = jnp.concatenate([src, jnp.full((npad_edges,), trash, i32)])
    dst_p = jnp.concatenate([dst, jnp.full((npad_edges,), trash, i32)])
    et_p = jnp.concatenate([et, jnp.zeros((npad_edges,), i32)])
    order = jnp.argsort(src_p)
    src_p = src_p[order]
    dst_p = dst_p[order]
    et_p = et_p[order]

    src32 = src_p.reshape(32, EPT32)
    dst32 = dst_p.reshape(32, EPT32)
    dhi = dst_p << 16
    packed_gcn = (dhi | src_p).reshape(16, EPT)
    packed_ggc = (dhi | (et_p * NPAD + src_p)).reshape(16, EPT)
    zrows = jnp.zeros((ROWS_PT, HALF), jnp.float32)

    sc_degrees, sc_propagate = _sc_kernels()
    deg2 = sc_degrees(src32, dst32)
    norms = _norm_call(deg2)
    norm_out = norms[0].reshape(NPAD, 1)
    norm_in = norms[1].reshape(NPAD, 1)

    feats_p = jnp.pad(feats, ((0, NPAD - N), (0, 0)))

    def gcn2_layer(h_p, W, beta):
        hn = _scale_call(h_p, norm_out).reshape(2 * NPAD, HALF)
        agg = sc_propagate(hn, packed_gcn, zrows)
        return _gcn2_call(agg.reshape(2, NPAD, HALF), feats_p, norm_in,
                          W, beta)

    def ggc_layer(h_p, Wl, bl, Wih, Whh, bih, bhh):
        for _ in range(2):
            tbl = _trans_call(h_p, Wl, bl).reshape(2 * NET * NPAD, HALF)
            a = sc_propagate(tbl, packed_ggc, zrows)
            h_p = _gru_call(a.reshape(2, NPAD, HALF), h_p, Wih, Whh,
                            bih, bhh)
        return h_p

    h1 = gcn2_layer(feats_p, W_gcn_0, BETA1)
    h2 = ggc_layer(h1, W_lin_0, b_lin_0, W_ih_0, W_hh_0, b_ih_0, b_hh_0)
    h3 = gcn2_layer(h2, W_gcn_1, BETA2)
    h4 = ggc_layer(h3, W_lin_1, b_lin_1, W_ih_1, W_hh_1, b_ih_1, b_hh_1)
    return (h1[:N], h2[:N], h3[:N], h4[:N])


# speedup vs baseline: 1.4662x; 1.4662x over previous
import functools
import math

import jax
import jax.numpy as jnp
from jax import lax
from jax.experimental import pallas as pl
from jax.experimental.pallas import tpu as pltpu
from jax.experimental.pallas import tpu_sc as plsc

N = 10000
D = 256
E = 160000
NET = 2
NPAD = 10240
HALF = 128
ET = E + N
K = 64
CPT = 168
EPT = CPT * K
EP = 16 * EPT
EPT32 = EP // 32
ROWS_PT = NPAD // 16
RED = 2 * NPAD // 16

ALPHA = 0.1
BETA1 = math.log(2.0)
BETA2 = math.log(1.5)

BN = 512
GRID = NPAD // BN


def _deg_body(src_ref, dst_ref, out_ref, acc, sbuf, dbuf, red, res, stage):
    c = lax.axis_index("c")
    s = lax.axis_index("s")
    w = c * 16 + s
    zeros16 = jnp.zeros((16,), jnp.int32)
    ones16 = jnp.ones((16,), jnp.int32)

    @pl.loop(0, 2 * NPAD // 16)
    def _zero(i):
        acc[pl.ds(i * 16, 16)] = zeros16

    pltpu.sync_copy(src_ref.at[w], sbuf)
    pltpu.sync_copy(dst_ref.at[w], dbuf)

    @pl.loop(0, EPT32 // 16)
    def _count(i):
        sv = sbuf[pl.ds(i * 16, 16)]
        plsc.addupdate_scatter(acc, [sv], ones16)
        dv = dbuf[pl.ds(i * 16, 16)] + NPAD
        plsc.addupdate_scatter(acc, [dv], ones16)

    pltpu.sync_copy(acc, stage.at[s])
    plsc.subcore_barrier()
    for k in range(16):
        pltpu.sync_copy(stage.at[k, pl.ds(s * RED, RED)], red.at[k])

    @pl.loop(0, RED // 16)
    def _reduce(j):
        tot = red[0, pl.ds(j * 16, 16)]
        for k in range(1, 16):
            tot = tot + red[k, pl.ds(j * 16, 16)]
        res[pl.ds(j * 16, 16)] = tot

    pltpu.sync_copy(res, out_ref.at[c, pl.ds(s * RED, RED)])


def _prop_body(table_ref, packed_ref, zeros_ref, out_ref,
               pbuf, gixs, dixs, rowbufs, acc, sems):
    c = lax.axis_index("c")
    s = lax.axis_index("s")
    coff = c * (table_ref.shape[0] // 2)
    pltpu.sync_copy(zeros_ref, acc.at[pl.ds(s * ROWS_PT, ROWS_PT)])
    pltpu.sync_copy(packed_ref.at[s], pbuf)
    plsc.subcore_barrier()

    def unpack(i, gb, db):
        for j in range(K // 16):
            pv = pbuf[pl.ds(i * K + j * 16, 16)]
            gb[pl.ds(j * 16, 16)] = (pv & 0xFFFF) + coff
            db[pl.ds(j * 16, 16)] = lax.shift_right_logical(pv, 16)

    for b in range(4):
        unpack(b, gixs[b], dixs[b])
        pltpu.async_copy(table_ref.at[gixs[b]], rowbufs[b], sems[b])

    def drain_scatter(b):
        pltpu.make_async_copy(table_ref.at[gixs[b]], rowbufs[b],
                              sems[b]).wait()
        pltpu.sync_copy(rowbufs[b], acc.at[dixs[b]], add=True)

    @pl.loop(0, CPT // 4 - 1)
    def _edges(k):
        for b in range(4):
            drain_scatter(b)
            nxt = 4 * k + 4 + b
            unpack(nxt, gixs[b], dixs[b])
            pltpu.async_copy(table_ref.at[gixs[b]], rowbufs[b], sems[b])

    for b in range(4):
        drain_scatter(b)

    plsc.subcore_barrier()
    pltpu.sync_copy(acc.at[pl.ds(s * ROWS_PT, ROWS_PT)],
                    out_ref.at[pl.ds(c * NPAD + s * ROWS_PT, ROWS_PT)])


@functools.lru_cache(maxsize=1)
def _sc_kernels():
    mesh = plsc.VectorSubcoreMesh(core_axis_name="c", subcore_axis_name="s")
    params = pltpu.CompilerParams(needs_layout_passes=False)
    deg = pl.kernel(
        _deg_body,
        out_type=jax.ShapeDtypeStruct((2, 2 * NPAD), jnp.int32),
        mesh=mesh,
        compiler_params=params,
        scratch_types=[
            pltpu.VMEM((2 * NPAD,), jnp.int32),
            pltpu.VMEM((EPT32,), jnp.int32),
            pltpu.VMEM((EPT32,), jnp.int32),
            pltpu.VMEM((16, RED), jnp.int32),
            pltpu.VMEM((RED,), jnp.int32),
            pltpu.VMEM_SHARED((16, 2 * NPAD), jnp.int32),
        ],
    )
    prop = pl.kernel(
        _prop_body,
        out_type=jax.ShapeDtypeStruct((2 * NPAD, HALF), jnp.float32),
        mesh=mesh,
        compiler_params=params,
        scratch_types=[
            pltpu.VMEM((EPT,), jnp.int32),
            [pltpu.VMEM((K,), jnp.int32) for _ in range(4)],
            [pltpu.VMEM((K,), jnp.int32) for _ in range(4)],
            [pltpu.VMEM((K, HALF), jnp.float32) for _ in range(4)],
            pltpu.VMEM_SHARED((NPAD, HALF), jnp.float32),
            [pltpu.SemaphoreType.DMA for _ in range(4)],
        ],
    )
    return deg, prop


def _norm_body(d_ref, out_ref):
    t = (d_ref[0] + d_ref[1]).astype(jnp.float32)
    r = lax.rsqrt(jnp.maximum(t, 1.0))
    out_ref[0] = r[:80]
    out_ref[1] = r[80:]


def _norm_call(deg2):
    return pl.pallas_call(
        _norm_body,
        out_shape=jax.ShapeDtypeStruct((2, 80, 128), jnp.float32),
    )(deg2.reshape(2, 160, 128))


def _scale_body(h_ref, n_ref, out_ref):
    h = h_ref[...]
    nv = n_ref[...]
    out_ref[0] = h[:, :HALF] * nv
    out_ref[1] = h[:, HALF:] * nv


def _scale_call(h_p, norm):
    return pl.pallas_call(
        _scale_body,
        grid=(GRID,),
        in_specs=[pl.BlockSpec((BN, D), lambda i: (i, 0)),
                  pl.BlockSpec((BN, 1), lambda i: (i, 0))],
        out_specs=pl.BlockSpec((2, BN, HALF), lambda i: (0, i, 0)),
        out_shape=jax.ShapeDtypeStruct((2, NPAD, HALF), jnp.float32),
    )(h_p, norm)


def _gcn2_body(a_ref, h0_ref, n_ref, w_ref, out_ref, *, beta):
    a = jnp.concatenate([a_ref[0], a_ref[1]], axis=1)
    hp = a * n_ref[...]
    f = (1.0 - ALPHA) * hp + ALPHA * h0_ref[...]
    rst = (1.0 - beta) * f + beta * jnp.dot(
        f, w_ref[...], preferred_element_type=jnp.float32)
    out_ref[...] = jnp.where(rst > 0, rst, jnp.exp(jnp.minimum(rst, 0.0)) - 1.0)


def _gcn2_call(agg3, h0_p, norm_in, W, beta):
    return pl.pallas_call(
        functools.partial(_gcn2_body, beta=beta),
        grid=(GRID,),
        in_specs=[pl.BlockSpec((2, BN, HALF), lambda i: (0, i, 0)),
                  pl.BlockSpec((BN, D), lambda i: (i, 0)),
                  pl.BlockSpec((BN, 1), lambda i: (i, 0)),
                  pl.BlockSpec((D, D), lambda i: (0, 0))],
        out_specs=pl.BlockSpec((BN, D), lambda i: (i, 0)),
        out_shape=jax.ShapeDtypeStruct((NPAD, D), jnp.float32),
    )(agg3, h0_p, norm_in, W)


def _trans_body(h_ref, w_ref, b_ref, out_ref):
    h = h_ref[...]
    for t in range(NET):
        m = lax.dot_general(h, w_ref[t], (((1,), (1,)), ((), ())),
                            preferred_element_type=jnp.float32) + b_ref[t]
        out_ref[0, t] = m[:, :HALF]
        out_ref[1, t] = m[:, HALF:]


def _trans_call(h_p, Wl, bl):
    return pl.pallas_call(
        _trans_body,
        grid=(GRID,),
        in_specs=[pl.BlockSpec((BN, D), lambda i: (i, 0)),
                  pl.BlockSpec((NET, D, D), lambda i: (0, 0, 0)),
                  pl.BlockSpec((NET, 1, D), lambda i: (0, 0, 0))],
        out_specs=pl.BlockSpec((2, NET, BN, HALF), lambda i: (0, 0, i, 0)),
        out_shape=jax.ShapeDtypeStruct((2, NET, NPAD, HALF), jnp.float32),
    )(h_p, Wl, bl.reshape(NET, 1, D))


def _gru_body(a_ref, h_ref, wih_ref, whh_ref, bih_ref, bhh_ref, out_ref):
    a = jnp.concatenate([a_ref[0], a_ref[1]], axis=1)
    h = h_ref[...]
    gi = lax.dot_general(a, wih_ref[...], (((1,), (1,)), ((), ())),
                         preferred_element_type=jnp.float32) + bih_ref[...]
    gh = lax.dot_general(h, whh_ref[...], (((1,), (1,)), ((), ())),
                         preferred_element_type=jnp.float32) + bhh_ref[...]
    r = jax.nn.sigmoid(gi[:, :D] + gh[:, :D])
    z = jax.nn.sigmoid(gi[:, D:2 * D] + gh[:, D:2 * D])
    g = jnp.tanh(gi[:, 2 * D:] + r * gh[:, 2 * D:])
    out_ref[...] = (1.0 - z) * g + z * h


def _gru_call(a3, h_p, Wih, Whh, bih, bhh):
    return pl.pallas_call(
        _gru_body,
        grid=(GRID,),
        in_specs=[pl.BlockSpec((2, BN, HALF), lambda i: (0, i, 0)),
                  pl.BlockSpec((BN, D), lambda i: (i, 0)),
                  pl.BlockSpec((3 * D, D), lambda i: (0, 0)),
                  pl.BlockSpec((3 * D, D), lambda i: (0, 0)),
                  pl.BlockSpec((1, 3 * D), lambda i: (0, 0)),
                  pl.BlockSpec((1, 3 * D), lambda i: (0, 0))],
        out_specs=pl.BlockSpec((BN, D), lambda i: (i, 0)),
        out_shape=jax.ShapeDtypeStruct((NPAD, D), jnp.float32),
    )(a3, h_p, Wih, Whh, bih.reshape(1, 3 * D), bhh.reshape(1, 3 * D))


def kernel(feats, edge_index, etypes, W_gcn_0, W_lin_0, b_lin_0, W_ih_0,
           W_hh_0, b_ih_0, b_hh_0, W_gcn_1, W_lin_1, b_lin_1, W_ih_1,
           W_hh_1, b_ih_1, b_hh_1):
    i32 = jnp.int32
    loop = jnp.arange(N, dtype=i32)
    src = jnp.concatenate([edge_index[0], loop])
    dst = jnp.concatenate([edge_index[1], loop])
    et = jnp.concatenate([etypes, jnp.zeros((N,), dtype=etypes.dtype)])
    npad_edges = EP - ET
    trash = NPAD - 1
    src_p = jnp.concatenate([src, jnp.full((npad_edges,), trash, i32)])
    dst_p = jnp.concatenate([dst, jnp.full((npad_edges,), trash, i32)])
    et_p = jnp.concatenate([et, jnp.zeros((npad_edges,), i32)])

    src32 = src_p.reshape(32, EPT32)
    dst32 = dst_p.reshape(32, EPT32)
    dhi = dst_p << 16
    packed_gcn = (dhi | src_p).reshape(16, EPT)
    packed_ggc = (dhi | (et_p * NPAD + src_p)).reshape(16, EPT)
    zrows = jnp.zeros((ROWS_PT, HALF), jnp.float32)

    sc_degrees, sc_propagate = _sc_kernels()
    deg2 = sc_degrees(src32, dst32)
    norms = _norm_call(deg2)
    norm_out = norms[0].reshape(NPAD, 1)
    norm_in = norms[1].reshape(NPAD, 1)

    feats_p = jnp.pad(feats, ((0, NPAD - N), (0, 0)))

    def gcn2_layer(h_p, W, beta):
        hn = _scale_call(h_p, norm_out).reshape(2 * NPAD, HALF)
        agg = sc_propagate(hn, packed_gcn, zrows)
        return _gcn2_call(agg.reshape(2, NPAD, HALF), feats_p, norm_in,
                          W, beta)

    def ggc_layer(h_p, Wl, bl, Wih, Whh, bih, bhh):
        for _ in range(2):
            tbl = _trans_call(h_p, Wl, bl).reshape(2 * NET * NPAD, HALF)
            a = sc_propagate(tbl, packed_ggc, zrows)
            h_p = _gru_call(a.reshape(2, NPAD, HALF), h_p, Wih, Whh,
                            bih, bhh)
        return h_p

    h1 = gcn2_layer(feats_p, W_gcn_0, BETA1)
    h2 = ggc_layer(h1, W_lin_0, b_lin_0, W_ih_0, W_hh_0, b_ih_0, b_hh_0)
    h3 = gcn2_layer(h2, W_gcn_1, BETA2)
    h4 = ggc_layer(h3, W_lin_1, b_lin_1, W_ih_1, W_hh_1, b_ih_1, b_hh_1)
    return (h1[:N], h2[:N], h3[:N], h4[:N])
